# baseline (speedup 1.0000x reference)
import functools

import jax
import jax.numpy as jnp
from jax import lax
from jax.experimental import pallas as pl
from jax.experimental.pallas import tpu as pltpu
from jax.experimental.pallas import tpu_sc as plsc

_MIN_VAL = 0
_VOCAB = 1_000_000
_EMBED_DIM = 16
_BATCH = 16384

_INFO = plsc.get_sparse_core_info()
_NC, _NS = _INFO.num_cores, _INFO.num_subcores
_NW = _NC * _NS
_CHUNK = 128
_B_PER_W = _BATCH // _NW
_CHUNKS_PER_W = _B_PER_W // _CHUNK

_mesh = plsc.VectorSubcoreMesh(core_axis_name="c", subcore_axis_name="s")


@functools.partial(
    pl.kernel,
    mesh=_mesh,
    compiler_params=pltpu.CompilerParams(use_tc_tiling_on_sc=False),
    out_type=jax.ShapeDtypeStruct((_BATCH, _EMBED_DIM), jnp.float32),
    scratch_types=[
        pltpu.VMEM((_CHUNKS_PER_W, _CHUNK), jnp.int32),
        pltpu.VMEM((_B_PER_W, _EMBED_DIM), jnp.float32),
        pltpu.SemaphoreType.DMA,
    ],
)
def _embed_lookup(idx_hbm, table_hbm, out_hbm, idx_v, rows_v, sem):
    wid = lax.axis_index("s") * _NC + lax.axis_index("c")
    row_base = wid * _CHUNKS_PER_W
    pltpu.sync_copy(idx_hbm.at[pl.ds(row_base, _CHUNKS_PER_W)], idx_v)
    copies = []
    for j in range(_CHUNKS_PER_W):
        copies.append(
            pltpu.async_copy(
                table_hbm.at[idx_v.at[j]],
                rows_v.at[pl.ds(j * _CHUNK, _CHUNK)],
                sem,
            )
        )
    for c in copies:
        c.wait()
    pltpu.sync_copy(rows_v, out_hbm.at[pl.ds(wid * _B_PER_W, _B_PER_W)])


def kernel(value, table):
    idx = value.reshape(_BATCH // _CHUNK, _CHUNK)
    return _embed_lookup(idx, table)


# speedup vs baseline: 4.9024x; 4.9024x over previous
import functools
import jax
import jax.numpy as jnp
from jax import lax
from jax.experimental import pallas as pl
from jax.experimental.pallas import tpu as pltpu
from jax.experimental.pallas import tpu_sc as plsc

_D = 16
_B = 16384
_NW = 32
_BPW = _B // _NW
_G = 16
_NG = _BPW // _G

_mesh = plsc.VectorSubcoreMesh(core_axis_name="c", subcore_axis_name="s")


@functools.partial(
    pl.kernel,
    mesh=_mesh,
    compiler_params=pltpu.CompilerParams(needs_layout_passes=False),
    out_type=jax.ShapeDtypeStruct((_D, _B), jnp.float32),
    scratch_types=[
        pltpu.VMEM((_BPW,), jnp.int32),
        pltpu.VMEM((_D, _G * 128), jnp.float32),
        pltpu.VMEM((_D, _BPW), jnp.float32),
        pltpu.SemaphoreType.DMA,
    ],
)
def _lookup(idx_hbm, table_t_hbm, out_hbm, idx_v, tiles, colbuf, sem):
    wid = lax.axis_index("s") * 2 + lax.axis_index("c")
    base = wid * _BPW
    pltpu.sync_copy(idx_hbm.at[pl.ds(base, _BPW)], idx_v)
    rows = lax.iota(jnp.int32, 16)

    def group(g, carry):
        vec = idx_v[pl.ds(g * _G, _G)]
        copies = []
        for l in range(_G):
            v = vec[l]
            cal = pl.multiple_of((v >> 7) * 128, 128)
            copies.append(
                pltpu.async_copy(
                    table_t_hbm.at[:, pl.ds(cal, 128)],
                    tiles.at[:, pl.ds(l * 128, 128)],
                    sem,
                )
            )
        for c in copies:
            c.wait()
        for l in range(_G):
            v = vec[l]
            w = jnp.full((16,), l * 128 + (v & 127), jnp.int32)
            emb = plsc.load_gather(tiles, [rows, w])
            j = jnp.full((16,), g * _G + l, jnp.int32)
            plsc.store_scatter(colbuf, [rows, j], emb)
        return carry

    lax.fori_loop(0, _NG, group, 0)
    pltpu.sync_copy(colbuf, out_hbm.at[:, pl.ds(base, _BPW)])


def kernel(value, table):
    table_t = jnp.swapaxes(table, 0, 1)
    out_t = _lookup(value, table_t)
    return jnp.swapaxes(out_t, 0, 1)


# speedup vs baseline: 5.8810x; 1.1996x over previous
import functools
import jax
import jax.numpy as jnp
from jax import lax
from jax.experimental import pallas as pl
from jax.experimental.pallas import tpu as pltpu
from jax.experimental.pallas import tpu_sc as plsc

_D = 16
_B = 16384
_NW = 32
_BPW = _B // _NW
_G = 16
_NG = _BPW // _G
_SLOT = _G * 128

_mesh = plsc.VectorSubcoreMesh(core_axis_name="c", subcore_axis_name="s")


@functools.partial(
    pl.kernel,
    mesh=_mesh,
    compiler_params=pltpu.CompilerParams(needs_layout_passes=False),
    out_type=jax.ShapeDtypeStruct((_D, _B), jnp.float32),
    scratch_types=[
        pltpu.VMEM((_BPW,), jnp.int32),
        pltpu.VMEM((_D, 2 * _SLOT), jnp.float32),
        pltpu.VMEM((_D, _BPW), jnp.float32),
        pltpu.SemaphoreType.DMA,
        pltpu.SemaphoreType.DMA,
    ],
)
def _lookup(idx_hbm, table_t_hbm, out_hbm, idx_v, tiles, colbuf, sem0, sem1):
    wid = lax.axis_index("s") * 2 + lax.axis_index("c")
    base = wid * _BPW
    pltpu.sync_copy(idx_hbm.at[pl.ds(base, _BPW)], idx_v)
    rows = lax.iota(jnp.int32, 16)
    sems = [sem0, sem1]

    def fire(g, b):
        vec = idx_v[pl.ds(g * _G, _G)]
        for l in range(_G):
            v = vec[l]
            cal = pl.multiple_of((v >> 7) * 128, 128)
            pltpu.async_copy(
                table_t_hbm.at[:, pl.ds(cal, 128)],
                tiles.at[:, pl.ds(b * _SLOT + l * 128, 128)],
                sems[b],
            )

    def drain(b):
        pltpu.make_async_copy(
            table_t_hbm.at[:, pl.ds(0, _SLOT)],
            tiles.at[:, pl.ds(b * _SLOT, _SLOT)],
            sems[b],
        ).wait()

    def extract(g, b):
        vec = idx_v[pl.ds(g * _G, _G)]
        for l in range(_G):
            v = vec[l]
            w = jnp.full((16,), b * _SLOT + l * 128 + (v & 127), jnp.int32)
            emb = plsc.load_gather(tiles, [rows, w])
            j = jnp.full((16,), g * _G + l, jnp.int32)
            plsc.store_scatter(colbuf, [rows, j], emb)

    def body(g2, carry):
        g = g2 * 2
        fire(g + 1, 1)
        drain(0)
        extract(g, 0)

        @pl.when(g + 2 < _NG)
        def _():
            fire(g + 2, 0)

        drain(1)
        extract(g + 1, 1)
        return carry

    fire(0, 0)
    lax.fori_loop(0, _NG // 2, body, 0)
    pltpu.sync_copy(colbuf, out_hbm.at[:, pl.ds(base, _BPW)])


def kernel(value, table):
    table_t = jnp.swapaxes(table, 0, 1)
    out_t = _lookup(value, table_t)
    return jnp.swapaxes(out_t, 0, 1)


# speedup vs baseline: 6.3852x; 1.0857x over previous
import functools
import jax
import jax.numpy as jnp
from jax import lax
from jax.experimental import pallas as pl
from jax.experimental.pallas import tpu as pltpu
from jax.experimental.pallas import tpu_sc as plsc

_D = 16
_B = 16384
_NW = 32
_BPW = _B // _NW
_G = 16
_NG = _BPW // _G
_SLOT = _G * 128

_mesh = plsc.VectorSubcoreMesh(core_axis_name="c", subcore_axis_name="s")


@functools.partial(
    pl.kernel,
    mesh=_mesh,
    compiler_params=pltpu.CompilerParams(needs_layout_passes=False),
    out_type=jax.ShapeDtypeStruct((_D, _B), jnp.float32),
    scratch_types=[
        pltpu.VMEM((_BPW,), jnp.int32),
        pltpu.VMEM((_D, 3 * _SLOT), jnp.float32),
        pltpu.VMEM((_D, _BPW), jnp.float32),
        pltpu.SemaphoreType.DMA,
        pltpu.SemaphoreType.DMA,
        pltpu.SemaphoreType.DMA,
    ],
)
def _lookup(idx_hbm, table_t_hbm, out_hbm, idx_v, tiles, colbuf, sem0, sem1,
            sem2):
    wid = lax.axis_index("s") * 2 + lax.axis_index("c")
    base = wid * _BPW
    pltpu.sync_copy(idx_hbm.at[pl.ds(base, _BPW)], idx_v)
    rows = lax.iota(jnp.int32, 16)
    sems = [sem0, sem1, sem2]

    def fire(g, b):
        vec = idx_v[pl.ds(g * _G, _G)]
        for l in range(_G):
            v = vec[l]
            cal = pl.multiple_of((v >> 7) * 128, 128)
            pltpu.async_copy(
                table_t_hbm.at[:, pl.ds(cal, 128)],
                tiles.at[:, pl.ds(b * _SLOT + l * 128, 128)],
                sems[b],
            )

    def drain(b):
        pltpu.make_async_copy(
            table_t_hbm.at[:, pl.ds(0, _SLOT)],
            tiles.at[:, pl.ds(b * _SLOT, _SLOT)],
            sems[b],
        ).wait()

    def extract(g, b):
        vec = idx_v[pl.ds(g * _G, _G)]
        for l in range(_G):
            v = vec[l]
            w = jnp.full((16,), b * _SLOT + l * 128 + (v & 127), jnp.int32)
            emb = plsc.load_gather(tiles, [rows, w])
            j = jnp.full((16,), g * _G + l, jnp.int32)
            plsc.store_scatter(colbuf, [rows, j], emb)

    def body(k, carry):
        for j in range(3):
            g = k * 3 + j

            @pl.when(g + 2 < _NG)
            def _(g=g, j=j):
                fire(g + 2, (j + 2) % 3)

            @pl.when(g < _NG)
            def _(g=g, j=j):
                drain(j)
                extract(g, j)
        return carry

    fire(0, 0)
    fire(1, 1)
    lax.fori_loop(0, (_NG + 2) // 3, body, 0)
    pltpu.sync_copy(colbuf, out_hbm.at[:, pl.ds(base, _BPW)])


def kernel(value, table):
    table_t = jnp.swapaxes(table, 0, 1)
    out_t = _lookup(value, table_t)
    return jnp.swapaxes(out_t, 0, 1)
